# baseline (speedup 1.0000x reference)
import jax
import jax.numpy as jnp
from jax import lax
from jax.experimental import pallas as pl
from jax.experimental.pallas import tpu as pltpu

_C1, _C2, _K = 8, 32, 5
_H_IN = 28
_HW = _H_IN * _H_IN
_H1 = _H_IN - _K + 1
_HP1 = _H1 // 2
_H2 = _HP1 - _K + 1
_HP2 = _H2 // 2
_NB = 256
_M1 = _H1 * _H1 * _C1
_K2 = _HP1 * _HP1 * _C1
_M2 = _H2 * _H2 * _C2
_DT = jnp.bfloat16


def _pack_w1(w1, dtype):
    a = w1[:, 0]
    rr = jnp.arange(_H_IN)[None, :] - jnp.arange(_H1)[:, None]
    vr = (rr >= 0) & (rr < _K)
    rc = jnp.clip(rr, 0, _K - 1)
    t = a[:, rc, :]
    t = t[:, :, :, rc]
    t = jnp.where(vr[None, :, :, None, None]
                  & vr[None, None, None, :, :], t, 0.0)
    t = jnp.transpose(t, (3, 1, 0, 2, 4))
    return t.reshape(_M1, _HW).astype(dtype)


def _pack_w2(w2, dtype):
    rr = jnp.arange(_HP1)[None, :] - jnp.arange(_H2)[:, None]
    vr = (rr >= 0) & (rr < _K)
    rc = jnp.clip(rr, 0, _K - 1)
    t = w2[:, :, rc, :]
    t = t[:, :, :, :, rc]
    t = jnp.where(vr[None, None, :, :, None, None]
                  & vr[None, None, None, None, :, :], t, 0.0)
    t = jnp.transpose(t, (2, 4, 0, 3, 5, 1))
    return t.reshape(_M2, _K2).astype(dtype)


def _net_kernel(x_ref, w1_ref, b1_ref, w2_ref, b2_ref, wf_ref, bf_ref, o_ref):
    f32 = jnp.float32
    nb = o_ref.shape[-1]
    xv = x_ref[0].astype(_DT)

    _CW = 4 * _H1 * _C1
    p1_parts = []
    for k in range(_H1 // 4):
        a1 = lax.dot_general(w1_ref[k * _CW:(k + 1) * _CW, :], xv,
                             (((1,), (1,)), ((), ())),
                             preferred_element_type=f32)
        a1 = a1.reshape(2, 2, _HP1, 2, _C1, nb)
        m1 = jnp.maximum(jnp.maximum(a1[:, 0, :, 0], a1[:, 0, :, 1]),
                         jnp.maximum(a1[:, 1, :, 0], a1[:, 1, :, 1]))
        p1_parts.append(jnp.transpose(m1, (1, 0, 2, 3)))
    p1 = jnp.concatenate(p1_parts, axis=1)
    p1 = jnp.maximum(p1 + b1_ref[...], 0.0)
    p1f = p1.reshape(_K2, nb).astype(_DT)

    _CW2 = _M2 // 2
    s = None
    for k in range(2):
        a2 = lax.dot_general(w2_ref[k * _CW2:(k + 1) * _CW2, :], p1f,
                             (((1,), (0,)), ((), ())),
                             preferred_element_type=f32)
        a2 = a2.reshape(2, 2, _HP2, 2, _C2, nb)
        m2 = jnp.maximum(jnp.maximum(a2[:, 0, :, 0], a2[:, 0, :, 1]),
                         jnp.maximum(a2[:, 1, :, 0], a2[:, 1, :, 1]))
        r2 = jnp.maximum(m2 + b2_ref[...], 0.0)
        t = jnp.sum(r2.reshape(2 * _HP2, _C2, nb), axis=0)
        s = t if s is None else s + t
    feat = s * (1.0 / 16.0)

    y = jnp.dot(wf_ref[...], feat, preferred_element_type=f32) + bf_ref[...]
    o_ref[...] = y.astype(o_ref.dtype)


def kernel(x, w1, b1, w2, b2, wf, bf):
    B = x.shape[0]
    out_dim = wf.shape[0]
    nb = _NB
    n_blk = (B + nb - 1) // nb
    B_pad = n_blk * nb

    xs = x.reshape(B, _HW)
    if B_pad != B:
        xs = jnp.pad(xs, ((0, B_pad - B), (0, 0)))
    xk = xs.reshape(n_blk, nb, _HW)

    w1k = _pack_w1(w1, _DT)
    b1k = b1.reshape(_C1, 1)
    w2k = _pack_w2(w2, _DT)
    b2k = b2.reshape(_C2, 1)
    bfk = bf.reshape(out_dim, 1)

    flops = B_pad * (2 * _C1 * _H1 * _H1 * _K * _K
                     + 2 * _C2 * _H2 * _H2 * _C1 * _K * _K
                     + 2 * out_dim * _C2)
    bytes_accessed = B_pad * (_HW + out_dim) * 4

    yt = pl.pallas_call(
        _net_kernel,
        out_shape=jax.ShapeDtypeStruct((out_dim, B_pad), jnp.float32),
        grid_spec=pltpu.PrefetchScalarGridSpec(
            num_scalar_prefetch=0,
            grid=(n_blk,),
            in_specs=[
                pl.BlockSpec((1, nb, _HW), lambda g: (g, 0, 0)),
                pl.BlockSpec((_M1, _HW), lambda g: (0, 0)),
                pl.BlockSpec((_C1, 1), lambda g: (0, 0)),
                pl.BlockSpec((_M2, _K2), lambda g: (0, 0)),
                pl.BlockSpec((_C2, 1), lambda g: (0, 0)),
                pl.BlockSpec((out_dim, _C2), lambda g: (0, 0)),
                pl.BlockSpec((out_dim, 1), lambda g: (0, 0)),
            ],
            out_specs=pl.BlockSpec((out_dim, nb), lambda g: (0, g)),
        ),
        compiler_params=pltpu.CompilerParams(
            dimension_semantics=("parallel",),
            vmem_limit_bytes=48 * 1024 * 1024),
        cost_estimate=pl.CostEstimate(flops=flops, transcendentals=0,
                                      bytes_accessed=bytes_accessed),
    )(xk, w1k, b1k, w2k, b2k, wf, bfk)
    return yt.T[:B]


# speedup vs baseline: 6.1007x; 6.1007x over previous
import jax
import jax.numpy as jnp
from jax import lax
from jax.experimental import pallas as pl
from jax.experimental.pallas import tpu as pltpu

_C1, _C2, _K = 8, 32, 5
_H_IN = 28
_HW = _H_IN * _H_IN
_H1 = _H_IN - _K + 1
_HP1 = _H1 // 2
_H2 = _HP1 - _K + 1
_HP2 = _H2 // 2
_NB = 256


def _pack_w1(w1):
    w1t = jnp.transpose(w1[:, 0, :, :], (0, 2, 1))
    ki = jnp.arange(32)[None, :]
    dr = jnp.arange(_H1)[:, None]
    ii = ki - dr
    valid = (ii >= 0) & (ii < _K)
    wb = w1t[:, :, jnp.clip(ii, 0, _K - 1)]
    wb = jnp.where(valid[None, None], wb, 0.0)
    wb = jnp.transpose(wb, (2, 0, 1, 3))
    return wb.reshape(_H1 * _C1, _K * 32)


def _pack_w2(w2):
    w2t = jnp.transpose(w2, (0, 3, 2, 1))
    i = jnp.arange(_HP1)[None, :]
    dr = jnp.arange(_H2)[:, None]
    ii = i - dr
    valid = (ii >= 0) & (ii < _K)
    wb = w2t[:, :, jnp.clip(ii, 0, _K - 1), :]
    wb = jnp.where(valid[None, None, :, :, None], wb, 0.0)
    wb = jnp.transpose(wb, (2, 0, 1, 3, 4))
    return wb.reshape(_H2 * _C2, _K * _HP1 * _C1)


def _net_kernel(x_ref, w1_ref, b1_ref, w2_ref, b2_ref, wf_ref, bf_ref, o_ref,
                xt_ref, p1_ref, p2_ref):
    f32 = jnp.float32
    nb = o_ref.shape[-1]

    t = lax.transpose(x_ref[0], (1, 0))
    for r in range(_H_IN):
        for c in range(_H_IN):
            xt_ref[r:r + 1, c * nb:(c + 1) * nb] = \
                t[r * _H_IN + c:r * _H_IN + c + 1, :]

    xv = xt_ref[...]
    ztail = jnp.zeros((4, _H1 * nb), f32)
    for kj in range(_K):
        p1_ref[kj * 32:kj * 32 + _H_IN, :] = xv[:, kj * nb:(kj + _H1) * nb]
        p1_ref[kj * 32 + _H_IN:(kj + 1) * 32, :] = ztail
    c1 = jnp.dot(w1_ref[...], p1_ref[...], preferred_element_type=f32)
    c1 = jnp.maximum(c1 + b1_ref[...], 0.0)

    c1 = c1.reshape(_HP1, 2, _C1, _H1 * nb)
    r1 = jnp.maximum(c1[:, 0], c1[:, 1])
    pool1 = jnp.concatenate(
        [jnp.maximum(r1[:, :, (2 * j) * nb:(2 * j + 1) * nb],
                     r1[:, :, (2 * j + 1) * nb:(2 * j + 2) * nb])
         for j in range(_HP1)], axis=-1)

    for kj in range(_K):
        p2_ref[kj * 96:(kj + 1) * 96, :] = (
            pool1[:, :, kj * nb:(kj + _H2) * nb].reshape(_HP1 * _C1, _H2 * nb))
    c2 = jnp.dot(w2_ref[...], p2_ref[...], preferred_element_type=f32)
    c2 = jnp.maximum(c2 + b2_ref[...], 0.0)

    c2 = c2.reshape(_HP2, 2, _C2, _H2 * nb)
    r2 = jnp.maximum(c2[:, 0], c2[:, 1])
    s = None
    for j2 in range(_HP2):
        tt = jnp.maximum(r2[:, :, (2 * j2) * nb:(2 * j2 + 1) * nb],
                         r2[:, :, (2 * j2 + 1) * nb:(2 * j2 + 2) * nb])
        s = tt if s is None else s + tt
    feat = (s[0] + s[1] + s[2] + s[3]) * (1.0 / 16.0)

    y = jnp.dot(wf_ref[...], feat, preferred_element_type=f32) + bf_ref[...]
    o_ref[...] = y.astype(o_ref.dtype)


def kernel(x, w1, b1, w2, b2, wf, bf):
    B = x.shape[0]
    out_dim = wf.shape[0]
    nb = _NB
    n_blk = (B + nb - 1) // nb
    B_pad = n_blk * nb

    xs = x.reshape(B, _HW)
    if B_pad != B:
        xs = jnp.pad(xs, ((0, B_pad - B), (0, 0)))
    xk = xs.reshape(n_blk, nb, _HW)

    w1k = _pack_w1(w1)
    b1k = jnp.tile(b1, _H1).reshape(_H1 * _C1, 1)
    w2k = _pack_w2(w2)
    b2k = jnp.tile(b2, _H2).reshape(_H2 * _C2, 1)
    bfk = bf.reshape(out_dim, 1)

    flops = B_pad * (2 * _C1 * _H1 * _H1 * _K * _K
                     + 2 * _C2 * _H2 * _H2 * _C1 * _K * _K
                     + 2 * out_dim * _C2)
    bytes_accessed = B_pad * (_HW + out_dim) * 4

    yt = pl.pallas_call(
        _net_kernel,
        out_shape=jax.ShapeDtypeStruct((out_dim, B_pad), jnp.float32),
        grid_spec=pltpu.PrefetchScalarGridSpec(
            num_scalar_prefetch=0,
            grid=(n_blk,),
            in_specs=[
                pl.BlockSpec((1, nb, _HW), lambda g: (g, 0, 0)),
                pl.BlockSpec((_H1 * _C1, _K * 32), lambda g: (0, 0)),
                pl.BlockSpec((_H1 * _C1, 1), lambda g: (0, 0)),
                pl.BlockSpec((_H2 * _C2, _K * _HP1 * _C1), lambda g: (0, 0)),
                pl.BlockSpec((_H2 * _C2, 1), lambda g: (0, 0)),
                pl.BlockSpec((out_dim, _C2), lambda g: (0, 0)),
                pl.BlockSpec((out_dim, 1), lambda g: (0, 0)),
            ],
            out_specs=pl.BlockSpec((out_dim, nb), lambda g: (0, g)),
            scratch_shapes=[
                pltpu.VMEM((_H_IN, _H_IN * nb), jnp.float32),
                pltpu.VMEM((_K * 32, _H1 * nb), jnp.float32),
                pltpu.VMEM((_K * _HP1 * _C1, _H2 * nb), jnp.float32),
            ],
        ),
        compiler_params=pltpu.CompilerParams(
            dimension_semantics=("arbitrary",),
            vmem_limit_bytes=48 * 1024 * 1024),
        cost_estimate=pl.CostEstimate(flops=flops, transcendentals=0,
                                      bytes_accessed=bytes_accessed),
    )(xk, w1k, b1k, w2k, b2k, wf, bfk)
    return yt.T[:B]


# speedup vs baseline: 13.2338x; 2.1692x over previous
import jax
import jax.numpy as jnp
from jax import lax
from jax.experimental import pallas as pl
from jax.experimental.pallas import tpu as pltpu

_C1, _C2, _K = 8, 32, 5
_H_IN = 28
_HW = _H_IN * _H_IN
_H1 = _H_IN - _K + 1
_HP1 = _H1 // 2
_H2 = _HP1 - _K + 1
_HP2 = _H2 // 2
_NB = 256


def _pack_w1(w1):
    w1t = jnp.transpose(w1[:, 0, :, :], (0, 2, 1))
    ki = jnp.arange(32)[None, :]
    dr = jnp.arange(_H1)[:, None]
    ii = ki - dr
    valid = (ii >= 0) & (ii < _K)
    wb = w1t[:, :, jnp.clip(ii, 0, _K - 1)]
    wb = jnp.where(valid[None, None], wb, 0.0)
    wb = jnp.transpose(wb, (2, 0, 1, 3))
    return wb.reshape(_H1 * _C1, _K * 32)


def _pack_w2(w2):
    w2t = jnp.transpose(w2, (0, 3, 2, 1))
    i = jnp.arange(_HP1)[None, :]
    dr = jnp.arange(_H2)[:, None]
    ii = i - dr
    valid = (ii >= 0) & (ii < _K)
    wb = w2t[:, :, jnp.clip(ii, 0, _K - 1), :]
    wb = jnp.where(valid[None, None, :, :, None], wb, 0.0)
    wb = jnp.transpose(wb, (2, 0, 1, 3, 4))
    return wb.reshape(_H2 * _C2, _K * _HP1 * _C1)


def _net_kernel(x_ref, w1_ref, b1_ref, w2_ref, b2_ref, wf_ref, bf_ref, o_ref,
                xt_ref, p1_ref, p2_ref):
    f32 = jnp.float32
    nb = o_ref.shape[0]

    t = lax.transpose(x_ref[0].reshape(nb, _HW), (1, 0))
    for r in range(_H_IN):
        for c in range(_H_IN):
            xt_ref[r:r + 1, c * nb:(c + 1) * nb] = \
                t[r * _H_IN + c:r * _H_IN + c + 1, :]

    xv = xt_ref[...]
    ztail = jnp.zeros((4, _H1 * nb), f32)
    for kj in range(_K):
        p1_ref[kj * 32:kj * 32 + _H_IN, :] = xv[:, kj * nb:(kj + _H1) * nb]
        p1_ref[kj * 32 + _H_IN:(kj + 1) * 32, :] = ztail
    c1 = jnp.dot(w1_ref[...], p1_ref[...], preferred_element_type=f32)
    c1 = jnp.maximum(c1 + b1_ref[...], 0.0)

    c1 = c1.reshape(_HP1, 2, _C1, _H1 * nb)
    r1 = jnp.maximum(c1[:, 0], c1[:, 1])
    pool1 = jnp.concatenate(
        [jnp.maximum(r1[:, :, (2 * j) * nb:(2 * j + 1) * nb],
                     r1[:, :, (2 * j + 1) * nb:(2 * j + 2) * nb])
         for j in range(_HP1)], axis=-1)

    for kj in range(_K):
        p2_ref[kj * 96:(kj + 1) * 96, :] = (
            pool1[:, :, kj * nb:(kj + _H2) * nb].reshape(_HP1 * _C1, _H2 * nb))
    c2 = jnp.dot(w2_ref[...], p2_ref[...], preferred_element_type=f32)
    c2 = jnp.maximum(c2 + b2_ref[...], 0.0)

    c2 = c2.reshape(_HP2, 2, _C2, _H2 * nb)
    r2 = jnp.maximum(c2[:, 0], c2[:, 1])
    s = None
    for j2 in range(_HP2):
        tt = jnp.maximum(r2[:, :, (2 * j2) * nb:(2 * j2 + 1) * nb],
                         r2[:, :, (2 * j2 + 1) * nb:(2 * j2 + 2) * nb])
        s = tt if s is None else s + tt
    feat = (s[0] + s[1] + s[2] + s[3]) * (1.0 / 16.0)

    y = lax.dot_general(feat, wf_ref[...], (((0,), (1,)), ((), ())),
                        preferred_element_type=f32)
    o_ref[...] = (y + bf_ref[...]).astype(o_ref.dtype)


def kernel(x, w1, b1, w2, b2, wf, bf):
    B = x.shape[0]
    out_dim = wf.shape[0]
    nb = _NB
    n_blk = (B + nb - 1) // nb
    B_pad = n_blk * nb

    xs = x[:, 0]
    if B_pad != B:
        xs = jnp.pad(xs, ((0, B_pad - B), (0, 0), (0, 0)))
    xk = xs.reshape(n_blk, nb, _H_IN, _H_IN)

    w1k = _pack_w1(w1)
    b1k = jnp.tile(b1, _H1).reshape(_H1 * _C1, 1)
    w2k = _pack_w2(w2)
    b2k = jnp.tile(b2, _H2).reshape(_H2 * _C2, 1)
    bfk = bf.reshape(1, out_dim)

    flops = B_pad * (2 * _C1 * _H1 * _H1 * _K * _K
                     + 2 * _C2 * _H2 * _H2 * _C1 * _K * _K
                     + 2 * out_dim * _C2)
    bytes_accessed = B_pad * (_HW + out_dim) * 4

    yt = pl.pallas_call(
        _net_kernel,
        out_shape=jax.ShapeDtypeStruct((B_pad, out_dim), jnp.float32),
        grid_spec=pltpu.PrefetchScalarGridSpec(
            num_scalar_prefetch=0,
            grid=(n_blk,),
            in_specs=[
                pl.BlockSpec((1, nb, _H_IN, _H_IN), lambda g: (g, 0, 0, 0)),
                pl.BlockSpec((_H1 * _C1, _K * 32), lambda g: (0, 0)),
                pl.BlockSpec((_H1 * _C1, 1), lambda g: (0, 0)),
                pl.BlockSpec((_H2 * _C2, _K * _HP1 * _C1), lambda g: (0, 0)),
                pl.BlockSpec((_H2 * _C2, 1), lambda g: (0, 0)),
                pl.BlockSpec((out_dim, _C2), lambda g: (0, 0)),
                pl.BlockSpec((1, out_dim), lambda g: (0, 0)),
            ],
            out_specs=pl.BlockSpec((nb, out_dim), lambda g: (g, 0)),
            scratch_shapes=[
                pltpu.VMEM((_H_IN, _H_IN * nb), jnp.float32),
                pltpu.VMEM((_K * 32, _H1 * nb), jnp.float32),
                pltpu.VMEM((_K * _HP1 * _C1, _H2 * nb), jnp.float32),
            ],
        ),
        compiler_params=pltpu.CompilerParams(
            dimension_semantics=("arbitrary",),
            vmem_limit_bytes=48 * 1024 * 1024),
        cost_estimate=pl.CostEstimate(flops=flops, transcendentals=0,
                                      bytes_accessed=bytes_accessed),
    )(xk, w1k, b1k, w2k, b2k, wf, bfk)
    return yt[:B]


# speedup vs baseline: 13.2626x; 1.0022x over previous
import jax
import jax.numpy as jnp
from jax import lax
from jax.experimental import pallas as pl
from jax.experimental.pallas import tpu as pltpu

_C1, _C2, _K = 8, 32, 5
_H_IN = 28
_HW = _H_IN * _H_IN
_H1 = _H_IN - _K + 1
_HP1 = _H1 // 2
_H2 = _HP1 - _K + 1
_HP2 = _H2 // 2
_NB = 256


def _pack_w1(w1):
    w1t = jnp.transpose(w1[:, 0, :, :], (0, 2, 1))
    ki = jnp.arange(32)[None, :]
    dr = jnp.arange(_H1)[:, None]
    ii = ki - dr
    valid = (ii >= 0) & (ii < _K)
    wb = w1t[:, :, jnp.clip(ii, 0, _K - 1)]
    wb = jnp.where(valid[None, None], wb, 0.0)
    wb = jnp.transpose(wb, (2, 0, 1, 3))
    return wb.reshape(_H1 * _C1, _K * 32)


def _pack_w2(w2):
    w2t = jnp.transpose(w2, (0, 3, 2, 1))
    i = jnp.arange(_HP1)[None, :]
    dr = jnp.arange(_H2)[:, None]
    ii = i - dr
    valid = (ii >= 0) & (ii < _K)
    wb = w2t[:, :, jnp.clip(ii, 0, _K - 1), :]
    wb = jnp.where(valid[None, None, :, :, None], wb, 0.0)
    wb = jnp.transpose(wb, (2, 0, 1, 3, 4))
    return wb.reshape(_H2 * _C2, _K * _HP1 * _C1)


def _net_kernel(xa_ref, xb_ref, w1_ref, b1_ref, w2_ref, b2_ref, wf_ref,
                bf_ref, o_ref, xt_ref, p1_ref, p2_ref):
    f32 = jnp.float32
    nb = o_ref.shape[0]
    hb = nb // 2

    ta = lax.transpose(xa_ref[0, 0].reshape(hb, _HW), (1, 0))
    tb = lax.transpose(xb_ref[0, 0].reshape(hb, _HW), (1, 0))
    for r in range(_H_IN):
        for c in range(_H_IN):
            p = r * _H_IN + c
            xt_ref[r:r + 1, c * nb:c * nb + hb] = ta[p:p + 1, :]
            xt_ref[r:r + 1, c * nb + hb:(c + 1) * nb] = tb[p:p + 1, :]

    xv = xt_ref[...]
    ztail = jnp.zeros((4, _H1 * nb), f32)
    for kj in range(_K):
        p1_ref[kj * 32:kj * 32 + _H_IN, :] = xv[:, kj * nb:(kj + _H1) * nb]
        p1_ref[kj * 32 + _H_IN:(kj + 1) * 32, :] = ztail
    c1 = jnp.dot(w1_ref[...], p1_ref[...], preferred_element_type=f32)
    c1 = jnp.maximum(c1 + b1_ref[...], 0.0)

    c1 = c1.reshape(_HP1, 2, _C1, _H1 * nb)
    r1 = jnp.maximum(c1[:, 0], c1[:, 1])
    pool1 = jnp.concatenate(
        [jnp.maximum(r1[:, :, (2 * j) * nb:(2 * j + 1) * nb],
                     r1[:, :, (2 * j + 1) * nb:(2 * j + 2) * nb])
         for j in range(_HP1)], axis=-1)

    for kj in range(_K):
        p2_ref[kj * 96:(kj + 1) * 96, :] = (
            pool1[:, :, kj * nb:(kj + _H2) * nb].reshape(_HP1 * _C1, _H2 * nb))
    c2 = jnp.dot(w2_ref[...], p2_ref[...], preferred_element_type=f32)
    c2 = jnp.maximum(c2 + b2_ref[...], 0.0)

    c2 = c2.reshape(_HP2, 2, _C2, _H2 * nb)
    r2 = jnp.maximum(c2[:, 0], c2[:, 1])
    s = None
    for j2 in range(_HP2):
        tt = jnp.maximum(r2[:, :, (2 * j2) * nb:(2 * j2 + 1) * nb],
                         r2[:, :, (2 * j2 + 1) * nb:(2 * j2 + 2) * nb])
        s = tt if s is None else s + tt
    feat = (s[0] + s[1] + s[2] + s[3]) * (1.0 / 16.0)

    y = lax.dot_general(feat, wf_ref[...], (((0,), (1,)), ((), ())),
                        preferred_element_type=f32)
    o_ref[...] = (y + bf_ref[...]).astype(o_ref.dtype)


def kernel(x, w1, b1, w2, b2, wf, bf):
    B = x.shape[0]
    out_dim = wf.shape[0]
    nb = _NB
    n_blk = (B + nb - 1) // nb
    B_pad = n_blk * nb

    xs = x[:, 0]
    if B_pad != B:
        xs = jnp.pad(xs, ((0, B_pad - B), (0, 0), (0, 0)))
    xk = xs.reshape(n_blk, 2, nb // 2, _H_IN, _H_IN)

    w1k = _pack_w1(w1)
    b1k = jnp.tile(b1, _H1).reshape(_H1 * _C1, 1)
    w2k = _pack_w2(w2)
    b2k = jnp.tile(b2, _H2).reshape(_H2 * _C2, 1)
    bfk = bf.reshape(1, out_dim)

    flops = B_pad * (2 * _C1 * _H1 * _H1 * _K * _K
                     + 2 * _C2 * _H2 * _H2 * _C1 * _K * _K
                     + 2 * out_dim * _C2)
    bytes_accessed = B_pad * (_HW + out_dim) * 4

    yt = pl.pallas_call(
        _net_kernel,
        out_shape=jax.ShapeDtypeStruct((B_pad, out_dim), jnp.float32),
        grid_spec=pltpu.PrefetchScalarGridSpec(
            num_scalar_prefetch=0,
            grid=(n_blk,),
            in_specs=[
                pl.BlockSpec((1, 1, nb // 2, _H_IN, _H_IN),
                             lambda g: (g, 0, 0, 0, 0)),
                pl.BlockSpec((1, 1, nb // 2, _H_IN, _H_IN),
                             lambda g: (g, 1, 0, 0, 0)),
                pl.BlockSpec((_H1 * _C1, _K * 32), lambda g: (0, 0)),
                pl.BlockSpec((_H1 * _C1, 1), lambda g: (0, 0)),
                pl.BlockSpec((_H2 * _C2, _K * _HP1 * _C1), lambda g: (0, 0)),
                pl.BlockSpec((_H2 * _C2, 1), lambda g: (0, 0)),
                pl.BlockSpec((out_dim, _C2), lambda g: (0, 0)),
                pl.BlockSpec((1, out_dim), lambda g: (0, 0)),
            ],
            out_specs=pl.BlockSpec((nb, out_dim), lambda g: (g, 0)),
            scratch_shapes=[
                pltpu.VMEM((_H_IN, _H_IN * nb), jnp.float32),
                pltpu.VMEM((_K * 32, _H1 * nb), jnp.float32),
                pltpu.VMEM((_K * _HP1 * _C1, _H2 * nb), jnp.float32),
            ],
        ),
        compiler_params=pltpu.CompilerParams(
            dimension_semantics=("arbitrary",),
            vmem_limit_bytes=48 * 1024 * 1024),
        cost_estimate=pl.CostEstimate(flops=flops, transcendentals=0,
                                      bytes_accessed=bytes_accessed),
    )(xk, xk, w1k, b1k, w2k, b2k, wf, bfk)
    return yt[:B]
